# baseline (speedup 1.0000x reference)
import functools

import jax
import jax.numpy as jnp
from jax import lax
from jax.experimental import pallas as pl
from jax.experimental.pallas import tpu as pltpu
from jax.experimental.pallas import tpu_sc as plsc

_N = 100000
_H = 256
_HP = 128

_NC = 2
_NS = 16
_NW = _NC * _NS
_CH = 128
_NCH = 6
_PW = _CH * _NCH
_SEG = _NW * _PW
_NSEG = 1
_SC_ROWS = _NSEG * _SEG

_B = 2048
_SEG_NB = _SEG // _B
_NB_ALL = 49
_OH_NB = _NB_ALL - _NSEG * _SEG_NB


def _table_body(emb_ref, w1_ref, out_ref):
    out_ref[...] = lax.dot_general(
        emb_ref[...], w1_ref[...], (((1,), (1,)), ((), ())),
        preferred_element_type=jnp.float32)


_sc_mesh = plsc.VectorSubcoreMesh(core_axis_name="c", subcore_axis_name="s")


@functools.partial(
    pl.kernel,
    mesh=_sc_mesh,
    out_type=jax.ShapeDtypeStruct((_SEG, _HP), jnp.float32),
    scratch_types=[
        pltpu.VMEM((_CH,), jnp.int32),
        pltpu.VMEM((_CH,), jnp.int32),
        pltpu.VMEM((_CH,), jnp.int32),
        pltpu.VMEM((_CH,), jnp.int32),
        pltpu.VMEM((_CH,), jnp.int32),
        pltpu.VMEM((_CH,), jnp.int32),
        pltpu.VMEM((_PW, _HP), jnp.float32),
        pltpu.SemaphoreType.DMA,
        pltpu.SemaphoreType.DMA,
        pltpu.SemaphoreType.DMA,
        pltpu.SemaphoreType.DMA,
        pltpu.SemaphoreType.DMA,
        pltpu.SemaphoreType.DMA,
        pltpu.SemaphoreType.DMA,
        pltpu.SemaphoreType.DMA,
        pltpu.SemaphoreType.DMA,
    ],
)
def _sc_gather(x_hbm, table_hbm, out_hbm, i0, i1, i2, i3, i4, i5, rows_v,
               isem, g0, g1, g2, g3, g4, g5, wa, wb):
    wid = lax.axis_index("s") * _NC + lax.axis_index("c")
    base = wid * _PW
    idxs = (i0, i1, i2, i3, i4, i5)
    ic = [pltpu.async_copy(x_hbm.at[wid].at[j], idxs[j], isem)
          for j in range(_NCH)]
    gsem = (g0, g1, g2, g3, g4, g5)
    gc = [None] * _NCH
    half = _NCH // 2
    for j in range(_NCH):
        ic[j].wait()
        gc[j] = pltpu.async_copy(
            table_hbm.at[idxs[j]],
            rows_v.at[pl.ds(j * _CH, _CH)], gsem[j])
    for j in range(half):
        gc[j].wait()
    wca = pltpu.async_copy(
        rows_v.at[pl.ds(0, half * _CH)],
        out_hbm.at[pl.ds(base, half * _CH)], wa)
    for j in range(half, _NCH):
        gc[j].wait()
    wcb = pltpu.async_copy(
        rows_v.at[pl.ds(half * _CH, _PW - half * _CH)],
        out_hbm.at[pl.ds(base + half * _CH, _PW - half * _CH)], wb)
    wca.wait()
    wcb.wait()


def _onehot_body(x_ref, chi_ref, b_ref, table_ref, w2_ref, w3_ref, blin_ref,
                 w11_ref, v_ref, v1_ref):
    x = x_ref[0, 0, :]
    onehot = (lax.broadcasted_iota(jnp.int32, (_B, 128), 1)
              == x[:, None]).astype(jnp.bfloat16)
    xe = lax.dot_general(onehot, table_ref[...], (((1,), (0,)), ((), ())),
                         preferred_element_type=jnp.float32)
    t2 = lax.dot_general(chi_ref[...], w2_ref[...], (((1,), (1,)), ((), ())),
                         preferred_element_type=jnp.float32)
    t3 = lax.dot_general(b_ref[...], w3_ref[...], (((1,), (1,)), ((), ())),
                         preferred_element_type=jnp.float32)
    v_pre = xe + t2 + t3 + blin_ref[...]
    v = v_pre * jax.nn.sigmoid(v_pre)
    v_ref[...] = v
    v1_ref[...] = lax.dot_general(v, w11_ref[...], (((1,), (1,)), ((), ())),
                                  preferred_element_type=jnp.float32)


def _seg_body(g_ref, chi_ref, b_ref, w2_ref, w3_ref, blin_ref, w11_ref,
              vin_ref, v1in_ref, v_ref, v1_ref):
    u = lax.bitcast_convert_type(g_ref[...], jnp.uint32)
    xe_lo = lax.bitcast_convert_type(u << 16, jnp.float32)
    xe_hi = lax.bitcast_convert_type(u & jnp.uint32(0xFFFF0000), jnp.float32)
    xe = jnp.concatenate([xe_lo, xe_hi], axis=1)
    t2 = lax.dot_general(chi_ref[...], w2_ref[...], (((1,), (1,)), ((), ())),
                         preferred_element_type=jnp.float32)
    t3 = lax.dot_general(b_ref[...], w3_ref[...], (((1,), (1,)), ((), ())),
                         preferred_element_type=jnp.float32)
    v_pre = xe + t2 + t3 + blin_ref[...]
    v = v_pre * jax.nn.sigmoid(v_pre)
    v_ref[...] = v
    v1_ref[...] = lax.dot_general(v, w11_ref[...], (((1,), (1,)), ((), ())),
                                  preferred_element_type=jnp.float32)


_full_f32 = jax.ShapeDtypeStruct((_N, _H), jnp.float32)
_wspec = pl.BlockSpec((_H, _H), lambda i: (0, 0))
_bspec = pl.BlockSpec((1, _H), lambda i: (0, 0))
_any = pl.BlockSpec(memory_space=pl.ANY)


@jax.jit
def kernel(x, chi, b, emb, W_lin, b_lin, W_lin1):
    x = x.astype(jnp.int32)
    emb_pad = jnp.pad(emb, ((0, 128 - emb.shape[0]), (0, 0)))
    W1 = W_lin[:, :_H]
    W2 = W_lin[:, _H:2 * _H]
    W3 = W_lin[:, 2 * _H:]
    blin2 = b_lin.reshape(1, _H)

    table = pl.pallas_call(
        _table_body,
        out_shape=jax.ShapeDtypeStruct((128, _H), jnp.float32),
    )(emb_pad, W1)

    table_bf = table.astype(jnp.bfloat16)
    tu = lax.bitcast_convert_type(table_bf, jnp.uint16)
    table_packed = lax.bitcast_convert_type(
        tu[:, :_HP].astype(jnp.uint32)
        | (tu[:, _HP:].astype(jnp.uint32) << 16),
        jnp.float32)

    x_sc = x[:_SC_ROWS].reshape(_NSEG, _NW, _NCH, _CH)
    gs = [_sc_gather(x_sc[s], table_packed) for s in range(_NSEG)]

    oh_off = _NSEG * _SEG_NB
    x_oh = jnp.pad(x[_SC_ROWS:], (0, _NB_ALL * _B - _N)).reshape(
        _OH_NB, 1, _B)
    v, v1 = pl.pallas_call(
        _onehot_body,
        grid=(_OH_NB,),
        in_specs=[
            pl.BlockSpec((1, 1, _B), lambda i: (i, 0, 0)),
            pl.BlockSpec((_B, _H), lambda i: (oh_off + i, 0)),
            pl.BlockSpec((_B, _H), lambda i: (oh_off + i, 0)),
            pl.BlockSpec((128, _H), lambda i: (0, 0)),
            _wspec, _wspec, _bspec, _wspec,
        ],
        out_specs=[
            pl.BlockSpec((_B, _H), lambda i: (oh_off + i, 0)),
            pl.BlockSpec((_B, _H), lambda i: (oh_off + i, 0)),
        ],
        out_shape=[_full_f32, _full_f32],
        compiler_params=pltpu.CompilerParams(
            dimension_semantics=("parallel",)),
    )(x_oh, chi, b, table_bf, W2, W3, blin2, W_lin1)

    for s in range(_NSEG):
        off = s * _SEG_NB
        v, v1 = pl.pallas_call(
            _seg_body,
            grid=(_SEG_NB,),
            in_specs=[
                pl.BlockSpec((_B, _HP), lambda i: (i, 0)),
                pl.BlockSpec((_B, _H), lambda i, o=off: (o + i, 0)),
                pl.BlockSpec((_B, _H), lambda i, o=off: (o + i, 0)),
                _wspec, _wspec, _bspec, _wspec,
                _any, _any,
            ],
            out_specs=[
                pl.BlockSpec((_B, _H), lambda i, o=off: (o + i, 0)),
                pl.BlockSpec((_B, _H), lambda i, o=off: (o + i, 0)),
            ],
            out_shape=[_full_f32, _full_f32],
            input_output_aliases={7: 0, 8: 1},
            compiler_params=pltpu.CompilerParams(
                dimension_semantics=("parallel",)),
        )(gs[s], chi, b, W2, W3, blin2, W_lin1, v, v1)
    return (v, v1)


# speedup vs baseline: 1.0525x; 1.0525x over previous
import functools

import jax
import jax.numpy as jnp
from jax import lax
from jax.experimental import pallas as pl
from jax.experimental.pallas import tpu as pltpu
from jax.experimental.pallas import tpu_sc as plsc

_N = 100000
_H = 256
_HP = 128

_NC = 2
_NS = 16
_NW = _NC * _NS
_CH = 128
_NCH = 6
_PW = _CH * _NCH
_SEG = _NW * _PW
_NSEG = 1
_SC_ROWS = _NSEG * _SEG

_B = 4096
_SEG_NB = _SEG // _B
_NB_ALL = 25
_OH_NB = _NB_ALL - _NSEG * _SEG_NB


def _table_body(emb_ref, w1_ref, out_ref):
    out_ref[...] = lax.dot_general(
        emb_ref[...], w1_ref[...], (((1,), (1,)), ((), ())),
        preferred_element_type=jnp.float32)


_sc_mesh = plsc.VectorSubcoreMesh(core_axis_name="c", subcore_axis_name="s")


@functools.partial(
    pl.kernel,
    mesh=_sc_mesh,
    out_type=jax.ShapeDtypeStruct((_SEG, _HP), jnp.float32),
    scratch_types=[
        pltpu.VMEM((_CH,), jnp.int32),
        pltpu.VMEM((_CH,), jnp.int32),
        pltpu.VMEM((_CH,), jnp.int32),
        pltpu.VMEM((_CH,), jnp.int32),
        pltpu.VMEM((_CH,), jnp.int32),
        pltpu.VMEM((_CH,), jnp.int32),
        pltpu.VMEM((_PW, _HP), jnp.float32),
        pltpu.SemaphoreType.DMA,
        pltpu.SemaphoreType.DMA,
        pltpu.SemaphoreType.DMA,
        pltpu.SemaphoreType.DMA,
        pltpu.SemaphoreType.DMA,
        pltpu.SemaphoreType.DMA,
        pltpu.SemaphoreType.DMA,
        pltpu.SemaphoreType.DMA,
        pltpu.SemaphoreType.DMA,
    ],
)
def _sc_gather(x_hbm, table_hbm, out_hbm, i0, i1, i2, i3, i4, i5, rows_v,
               isem, g0, g1, g2, g3, g4, g5, wa, wb):
    wid = lax.axis_index("s") * _NC + lax.axis_index("c")
    base = wid * _PW
    idxs = (i0, i1, i2, i3, i4, i5)
    ic = [pltpu.async_copy(x_hbm.at[wid].at[j], idxs[j], isem)
          for j in range(_NCH)]
    gsem = (g0, g1, g2, g3, g4, g5)
    gc = [None] * _NCH
    half = _NCH // 2
    for j in range(_NCH):
        ic[j].wait()
        gc[j] = pltpu.async_copy(
            table_hbm.at[idxs[j]],
            rows_v.at[pl.ds(j * _CH, _CH)], gsem[j])
    for j in range(half):
        gc[j].wait()
    wca = pltpu.async_copy(
        rows_v.at[pl.ds(0, half * _CH)],
        out_hbm.at[pl.ds(base, half * _CH)], wa)
    for j in range(half, _NCH):
        gc[j].wait()
    wcb = pltpu.async_copy(
        rows_v.at[pl.ds(half * _CH, _PW - half * _CH)],
        out_hbm.at[pl.ds(base + half * _CH, _PW - half * _CH)], wb)
    wca.wait()
    wcb.wait()


def _onehot_body(x_ref, chi_ref, b_ref, table_ref, w2_ref, w3_ref, blin_ref,
                 w11_ref, v_ref, v1_ref):
    x = x_ref[0, 0, :]
    onehot = (lax.broadcasted_iota(jnp.int32, (_B, 128), 1)
              == x[:, None]).astype(jnp.bfloat16)
    xe = lax.dot_general(onehot, table_ref[...], (((1,), (0,)), ((), ())),
                         preferred_element_type=jnp.float32)
    t2 = lax.dot_general(chi_ref[...], w2_ref[...], (((1,), (1,)), ((), ())),
                         preferred_element_type=jnp.float32)
    t3 = lax.dot_general(b_ref[...], w3_ref[...], (((1,), (1,)), ((), ())),
                         preferred_element_type=jnp.float32)
    v_pre = xe + t2 + t3 + blin_ref[...]
    v = v_pre * jax.nn.sigmoid(v_pre)
    v_ref[...] = v
    v1_ref[...] = lax.dot_general(v, w11_ref[...], (((1,), (1,)), ((), ())),
                                  preferred_element_type=jnp.float32)


def _seg_body(g_ref, chi_ref, b_ref, w2_ref, w3_ref, blin_ref, w11_ref,
              vin_ref, v1in_ref, v_ref, v1_ref):
    u = lax.bitcast_convert_type(g_ref[...], jnp.uint32)
    xe_lo = lax.bitcast_convert_type(u << 16, jnp.float32)
    xe_hi = lax.bitcast_convert_type(u & jnp.uint32(0xFFFF0000), jnp.float32)
    xe = jnp.concatenate([xe_lo, xe_hi], axis=1)
    t2 = lax.dot_general(chi_ref[...], w2_ref[...], (((1,), (1,)), ((), ())),
                         preferred_element_type=jnp.float32)
    t3 = lax.dot_general(b_ref[...], w3_ref[...], (((1,), (1,)), ((), ())),
                         preferred_element_type=jnp.float32)
    v_pre = xe + t2 + t3 + blin_ref[...]
    v = v_pre * jax.nn.sigmoid(v_pre)
    v_ref[...] = v
    v1_ref[...] = lax.dot_general(v, w11_ref[...], (((1,), (1,)), ((), ())),
                                  preferred_element_type=jnp.float32)


_full_f32 = jax.ShapeDtypeStruct((_N, _H), jnp.float32)
_wspec = pl.BlockSpec((_H, _H), lambda i: (0, 0))
_bspec = pl.BlockSpec((1, _H), lambda i: (0, 0))
_any = pl.BlockSpec(memory_space=pl.ANY)


@jax.jit
def kernel(x, chi, b, emb, W_lin, b_lin, W_lin1):
    x = x.astype(jnp.int32)
    emb_pad = jnp.pad(emb, ((0, 128 - emb.shape[0]), (0, 0)))
    W1 = W_lin[:, :_H]
    W2 = W_lin[:, _H:2 * _H]
    W3 = W_lin[:, 2 * _H:]
    blin2 = b_lin.reshape(1, _H)

    table = pl.pallas_call(
        _table_body,
        out_shape=jax.ShapeDtypeStruct((128, _H), jnp.float32),
    )(emb_pad, W1)

    table_bf = table.astype(jnp.bfloat16)
    tu = lax.bitcast_convert_type(table_bf, jnp.uint16)
    table_packed = lax.bitcast_convert_type(
        tu[:, :_HP].astype(jnp.uint32)
        | (tu[:, _HP:].astype(jnp.uint32) << 16),
        jnp.float32)

    x_sc = x[:_SC_ROWS].reshape(_NSEG, _NW, _NCH, _CH)
    gs = [_sc_gather(x_sc[s], table_packed) for s in range(_NSEG)]

    oh_off = _NSEG * _SEG_NB
    x_oh = jnp.pad(x[_SC_ROWS:], (0, _NB_ALL * _B - _N)).reshape(
        _OH_NB, 1, _B)
    v, v1 = pl.pallas_call(
        _onehot_body,
        grid=(_OH_NB,),
        in_specs=[
            pl.BlockSpec((1, 1, _B), lambda i: (i, 0, 0)),
            pl.BlockSpec((_B, _H), lambda i: (oh_off + i, 0)),
            pl.BlockSpec((_B, _H), lambda i: (oh_off + i, 0)),
            pl.BlockSpec((128, _H), lambda i: (0, 0)),
            _wspec, _wspec, _bspec, _wspec,
        ],
        out_specs=[
            pl.BlockSpec((_B, _H), lambda i: (oh_off + i, 0)),
            pl.BlockSpec((_B, _H), lambda i: (oh_off + i, 0)),
        ],
        out_shape=[_full_f32, _full_f32],
        compiler_params=pltpu.CompilerParams(
            dimension_semantics=("arbitrary",)),
    )(x_oh, chi, b, table_bf, W2, W3, blin2, W_lin1)

    for s in range(_NSEG):
        off = s * _SEG_NB
        v, v1 = pl.pallas_call(
            _seg_body,
            grid=(_SEG_NB,),
            in_specs=[
                pl.BlockSpec((_B, _HP), lambda i: (i, 0)),
                pl.BlockSpec((_B, _H), lambda i, o=off: (o + i, 0)),
                pl.BlockSpec((_B, _H), lambda i, o=off: (o + i, 0)),
                _wspec, _wspec, _bspec, _wspec,
                _any, _any,
            ],
            out_specs=[
                pl.BlockSpec((_B, _H), lambda i, o=off: (o + i, 0)),
                pl.BlockSpec((_B, _H), lambda i, o=off: (o + i, 0)),
            ],
            out_shape=[_full_f32, _full_f32],
            input_output_aliases={7: 0, 8: 1},
            compiler_params=pltpu.CompilerParams(
                dimension_semantics=("arbitrary",)),
        )(gs[s], chi, b, W2, W3, blin2, W_lin1, v, v1)
    return (v, v1)
